# baseline (speedup 1.0000x reference)
import jax
import jax.numpy as jnp
from jax.experimental import pallas as pl
from jax.experimental.pallas import tpu as pltpu


def _fused_kernel(x_ref, W1_ref, b1_ref, W2_ref, b2_ref, Wg_ref,
                  asrc_ref, adst_ref, Wd1_ref, bd1_ref, Wd2_ref, bd2_ref,
                  o_ref, hb_scr, s_scr, d_scr):
    p = pl.program_id(0)
    i = pl.program_id(1)
    B = x_ref.shape[0]

    @pl.when(p == 0)
    def _encoder():
        x = x_ref[...].astype(jnp.bfloat16)
        z = jnp.dot(x, W1_ref[...].astype(jnp.bfloat16),
                    preferred_element_type=jnp.float32) + b1_ref[...]
        z = jnp.maximum(z, 0.0).astype(jnp.bfloat16)
        obs = jnp.dot(z, W2_ref[...].astype(jnp.bfloat16),
                      preferred_element_type=jnp.float32) + b2_ref[...]
        h = jnp.dot(obs.astype(jnp.bfloat16), Wg_ref[...].astype(jnp.bfloat16),
                    preferred_element_type=jnp.float32)
        D = h.shape[1]
        col = jax.lax.broadcasted_iota(jnp.int32, h.shape, 1)
        hb_scr[pl.ds(i * B, B), :] = jnp.concatenate(
            [h.astype(jnp.bfloat16), (col == 0).astype(jnp.bfloat16)], axis=1)
        s_scr[pl.ds(i * B, B), :] = jnp.dot(h, asrc_ref[...],
                                            preferred_element_type=jnp.float32)
        d_scr[:, pl.ds(i * B, B)] = jax.lax.dot_general(
            adst_ref[...], h, (((1,), (1,)), ((), ())),
            preferred_element_type=jnp.float32)

    @pl.when(p == 1)
    def _attention_decoder():
        s = s_scr[pl.ds(i * B, B), :]
        d = d_scr[...]
        N = d.shape[1]
        dmax = jnp.max(d)
        C = jnp.exp(-0.8 * (s + dmax)).astype(jnp.bfloat16)
        E1 = jnp.exp(d - dmax).astype(jnp.bfloat16)
        E2 = jnp.exp(0.2 * (d - dmax)).astype(jnp.bfloat16)
        CH = N // 4
        nd = None
        for c in range(4):
            lo = c * CH
            uc = jnp.maximum(E1[:, lo:lo + CH], C * E2[:, lo:lo + CH])
            part = jnp.dot(uc, hb_scr[lo:lo + CH, :],
                           preferred_element_type=jnp.float32)
            nd = part if nd is None else nd + part
        D = nd.shape[1] // 2
        num = nd[:, :D]
        denom = nd[:, D:D + 1]
        comm = num / denom
        comm = jnp.where(comm > 0, comm, jnp.exp(comm) - 1.0)
        z = jnp.dot(comm, Wd1_ref[...], preferred_element_type=jnp.float32) + bd1_ref[...]
        z = jnp.maximum(z, 0.0)
        o_ref[...] = jnp.dot(z, Wd2_ref[...], preferred_element_type=jnp.float32) + bd2_ref[...]


def kernel(X, W1, b1, W2, b2, Wg, a_src, a_dst, Wd1, bd1, Wd2, bd2):
    x = X[0]
    N, D = x.shape
    H = W1.shape[1]
    O = Wd2.shape[1]
    B = 1024

    const = lambda p, i: (0, 0)
    out = pl.pallas_call(
        _fused_kernel,
        grid=(2, N // B),
        in_specs=[
            pl.BlockSpec((B, D), lambda p, i: (i * (1 - p), 0)),
            pl.BlockSpec((D, H), const),
            pl.BlockSpec((1, H), const),
            pl.BlockSpec((H, D), const),
            pl.BlockSpec((1, D), const),
            pl.BlockSpec((D, D), const),
            pl.BlockSpec((D, 1), const),
            pl.BlockSpec((1, D), const),
            pl.BlockSpec((D, H), const),
            pl.BlockSpec((1, H), const),
            pl.BlockSpec((H, O), const),
            pl.BlockSpec((1, O), const),
        ],
        out_specs=pl.BlockSpec((B, O), lambda p, i: (i, 0)),
        out_shape=jax.ShapeDtypeStruct((N, O), jnp.float32),
        scratch_shapes=[
            pltpu.VMEM((N, 2 * D), jnp.bfloat16),
            pltpu.VMEM((N, 1), jnp.float32),
            pltpu.VMEM((1, N), jnp.float32),
        ],
    )(x, W1, b1.reshape(1, H), W2, b2.reshape(1, D), Wg,
      a_src.reshape(D, 1), a_dst.reshape(1, D),
      Wd1, bd1.reshape(1, H), Wd2, bd2.reshape(1, O))
    return out


# speedup vs baseline: 1.0511x; 1.0511x over previous
import jax
import jax.numpy as jnp
from jax.experimental import pallas as pl
from jax.experimental.pallas import tpu as pltpu


def _fused_kernel(x_ref, W1_ref, b1_ref, W2_ref, b2_ref, Wg_ref,
                  asrc_ref, adst_ref, Wd1_ref, bd1_ref, Wd2_ref, bd2_ref,
                  o_ref, hb_scr, s_scr, d_scr):
    p = pl.program_id(0)
    i = pl.program_id(1)
    B = x_ref.shape[0]

    @pl.when(p == 0)
    def _encoder():
        x = x_ref[...].astype(jnp.bfloat16)
        z = jnp.dot(x, W1_ref[...].astype(jnp.bfloat16),
                    preferred_element_type=jnp.float32) + b1_ref[...]
        z = jnp.maximum(z, 0.0).astype(jnp.bfloat16)
        obs = jnp.dot(z, W2_ref[...].astype(jnp.bfloat16),
                      preferred_element_type=jnp.float32) + b2_ref[...]
        h = jnp.dot(obs.astype(jnp.bfloat16), Wg_ref[...].astype(jnp.bfloat16),
                    preferred_element_type=jnp.float32)
        D = h.shape[1]
        col = jax.lax.broadcasted_iota(jnp.int32, h.shape, 1)
        hb_scr[pl.ds(i * B, B), :] = jnp.concatenate(
            [h.astype(jnp.bfloat16), (col == 0).astype(jnp.bfloat16)], axis=1)
        s_scr[pl.ds(i * B, B), :] = jnp.dot(h, asrc_ref[...],
                                            preferred_element_type=jnp.float32)
        d_scr[:, pl.ds(i * B, B)] = jax.lax.dot_general(
            adst_ref[...], h, (((1,), (1,)), ((), ())),
            preferred_element_type=jnp.float32)

    @pl.when(p == 1)
    def _attention_decoder():
        s = s_scr[pl.ds(i * B, B), :]
        d = d_scr[...]
        N = d.shape[1]
        dmax = jnp.max(d)
        C = jnp.exp(-0.8 * (s + dmax)).astype(jnp.bfloat16)
        E1 = jnp.exp(d - dmax).astype(jnp.bfloat16)
        E2 = jnp.exp(0.2 * (d - dmax)).astype(jnp.bfloat16)
        CH = N // 4
        nd = None
        for c in range(4):
            lo = c * CH
            uc = jnp.maximum(E1[:, lo:lo + CH], C * E2[:, lo:lo + CH])
            part = jnp.dot(uc, hb_scr[lo:lo + CH, :],
                           preferred_element_type=jnp.float32)
            nd = part if nd is None else nd + part
        D = nd.shape[1] // 2
        num = nd[:, :D]
        denom = nd[:, D:D + 1]
        comm = num / denom
        comm = jnp.where(comm > 0, comm, jnp.exp(comm) - 1.0)
        z = jnp.dot(comm, Wd1_ref[...], preferred_element_type=jnp.float32) + bd1_ref[...]
        z = jnp.maximum(z, 0.0)
        o_ref[...] = jnp.dot(z, Wd2_ref[...], preferred_element_type=jnp.float32) + bd2_ref[...]


def kernel(X, W1, b1, W2, b2, Wg, a_src, a_dst, Wd1, bd1, Wd2, bd2):
    x = X[0]
    N, D = x.shape
    H = W1.shape[1]
    O = Wd2.shape[1]
    B = 2048

    const = lambda p, i: (0, 0)
    out = pl.pallas_call(
        _fused_kernel,
        grid=(2, N // B),
        in_specs=[
            pl.BlockSpec((B, D), lambda p, i: (i * (1 - p), 0)),
            pl.BlockSpec((D, H), const),
            pl.BlockSpec((1, H), const),
            pl.BlockSpec((H, D), const),
            pl.BlockSpec((1, D), const),
            pl.BlockSpec((D, D), const),
            pl.BlockSpec((D, 1), const),
            pl.BlockSpec((1, D), const),
            pl.BlockSpec((D, H), const),
            pl.BlockSpec((1, H), const),
            pl.BlockSpec((H, O), const),
            pl.BlockSpec((1, O), const),
        ],
        out_specs=pl.BlockSpec((B, O), lambda p, i: (i, 0)),
        out_shape=jax.ShapeDtypeStruct((N, O), jnp.float32),
        scratch_shapes=[
            pltpu.VMEM((N, 2 * D), jnp.bfloat16),
            pltpu.VMEM((N, 1), jnp.float32),
            pltpu.VMEM((1, N), jnp.float32),
        ],
    )(x, W1, b1.reshape(1, H), W2, b2.reshape(1, D), Wg,
      a_src.reshape(D, 1), a_dst.reshape(1, D),
      Wd1, bd1.reshape(1, H), Wd2, bd2.reshape(1, O))
    return out


# speedup vs baseline: 1.0700x; 1.0180x over previous
import jax
import jax.numpy as jnp
from jax.experimental import pallas as pl
from jax.experimental.pallas import tpu as pltpu


def _fused_kernel(x_ref, W1_ref, b1_ref, W2_ref, b2_ref, Wg_ref,
                  asrc_ref, adst_ref, Wd1_ref, bd1_ref, Wd2_ref, bd2_ref,
                  o_ref, hb_scr, s_scr, d_scr):
    p = pl.program_id(0)
    i = pl.program_id(1)
    B = x_ref.shape[0]

    @pl.when(p == 0)
    def _encoder():
        x = x_ref[...].astype(jnp.bfloat16)
        z = jnp.dot(x, W1_ref[...].astype(jnp.bfloat16),
                    preferred_element_type=jnp.float32) + b1_ref[...]
        z = jnp.maximum(z, 0.0).astype(jnp.bfloat16)
        obs = jnp.dot(z, W2_ref[...].astype(jnp.bfloat16),
                      preferred_element_type=jnp.float32) + b2_ref[...]
        h = jnp.dot(obs.astype(jnp.bfloat16), Wg_ref[...].astype(jnp.bfloat16),
                    preferred_element_type=jnp.float32)
        col = jax.lax.broadcasted_iota(jnp.int32, h.shape, 1)
        hb_scr[pl.ds(i * B, B), :] = jnp.concatenate(
            [h.astype(jnp.bfloat16), (col == 0).astype(jnp.bfloat16)], axis=1)
        s_scr[pl.ds(i * B, B), :] = jnp.dot(h, asrc_ref[...],
                                            preferred_element_type=jnp.float32)
        d_scr[:, pl.ds(i * B, B)] = jax.lax.dot_general(
            adst_ref[...], h, (((1,), (1,)), ((), ())),
            preferred_element_type=jnp.float32)

    @pl.when(p == 1)
    def _attention_decoder():
        s = s_scr[pl.ds(i * B, B), :]
        d = d_scr[...]
        N = d.shape[1]
        dmax = jnp.max(d)
        C = jnp.exp(-0.8 * (s + dmax)).astype(jnp.bfloat16)
        E1 = jnp.exp(d - dmax).astype(jnp.bfloat16)
        E2 = jnp.exp(0.2 * (d - dmax)).astype(jnp.bfloat16)
        CH = N // 4
        nd = None
        for c in range(4):
            lo = c * CH
            uc = jnp.maximum(E1[:, lo:lo + CH], C * E2[:, lo:lo + CH])
            part = jnp.dot(uc, hb_scr[lo:lo + CH, :],
                           preferred_element_type=jnp.float32)
            nd = part if nd is None else nd + part
        D = nd.shape[1] // 2
        num = nd[:, :D]
        denom = nd[:, D:D + 1]
        comm = num / denom
        comm = jnp.where(comm > 0, comm, jnp.exp(comm) - 1.0)
        z = jnp.dot(comm, Wd1_ref[...], preferred_element_type=jnp.float32) + bd1_ref[...]
        z = jnp.maximum(z, 0.0)
        o_ref[...] = jnp.dot(z, Wd2_ref[...], preferred_element_type=jnp.float32) + bd2_ref[...]


def kernel(X, W1, b1, W2, b2, Wg, a_src, a_dst, Wd1, bd1, Wd2, bd2):
    x = X[0]
    N, D = x.shape
    H = W1.shape[1]
    O = Wd2.shape[1]
    B = 2048

    const = lambda p, i: (0, 0)
    out = pl.pallas_call(
        _fused_kernel,
        grid=(2, N // B),
        in_specs=[
            pl.BlockSpec((B, D), lambda p, i: (i * (1 - p), 0)),
            pl.BlockSpec((D, H), const),
            pl.BlockSpec((1, H), const),
            pl.BlockSpec((H, D), const),
            pl.BlockSpec((1, D), const),
            pl.BlockSpec((D, D), const),
            pl.BlockSpec((D, 1), const),
            pl.BlockSpec((1, D), const),
            pl.BlockSpec((D, H), const),
            pl.BlockSpec((1, H), const),
            pl.BlockSpec((H, O), const),
            pl.BlockSpec((1, O), const),
        ],
        out_specs=pl.BlockSpec((B, O), lambda p, i: (i * p, 0)),
        out_shape=jax.ShapeDtypeStruct((N, O), jnp.float32),
        scratch_shapes=[
            pltpu.VMEM((N, 2 * D), jnp.bfloat16),
            pltpu.VMEM((N, 1), jnp.float32),
            pltpu.VMEM((1, N), jnp.float32),
        ],
    )(x, W1, b1.reshape(1, H), W2, b2.reshape(1, D), Wg,
      a_src.reshape(D, 1), a_dst.reshape(1, D),
      Wd1, bd1.reshape(1, H), Wd2, bd2.reshape(1, O))
    return out
